# Initial kernel scaffold; baseline (speedup 1.0000x reference)
import jax
import jax.numpy as jnp
from jax.experimental import pallas as pl


def kernel(x, W_gate, b_gate):
    raise NotImplementedError("write your pallas kernel here")


# speedup vs baseline: 1.4909x; 1.4909x over previous
import jax
import jax.numpy as jnp
from jax.experimental import pallas as pl
from jax.experimental.pallas import tpu as pltpu

_ROWS_PER_BLOCK = 512


def _router_block(x_ref, w_ref, b_ref, idx_ref, probs_ref):
    logits = jnp.dot(x_ref[...], w_ref[...], preferred_element_type=jnp.float32)
    logits = logits + b_ref[...]
    r, e = logits.shape
    col = jax.lax.broadcasted_iota(jnp.int32, (r, e), 1)
    m1 = jnp.max(logits, axis=1, keepdims=True)
    i1 = jnp.min(jnp.where(logits == m1, col, e), axis=1, keepdims=True)
    masked = jnp.where(col == i1, -jnp.inf, logits)
    m2 = jnp.max(masked, axis=1, keepdims=True)
    i2 = jnp.min(jnp.where(masked == m2, col, e), axis=1, keepdims=True)
    idx_ref[...] = jnp.concatenate([i1, i2], axis=1)
    ex = jnp.exp(m2 - m1)
    denom = 1.0 + ex
    probs_ref[...] = jnp.concatenate([1.0 / denom, ex / denom], axis=1)


def kernel(x, W_gate, b_gate):
    n, d = x.shape
    e = W_gate.shape[1]
    r = _ROWS_PER_BLOCK
    idx, probs = pl.pallas_call(
        _router_block,
        grid=(n // r,),
        in_specs=[
            pl.BlockSpec((r, d), lambda i: (i, 0)),
            pl.BlockSpec((d, e), lambda i: (0, 0)),
            pl.BlockSpec((1, e), lambda i: (0, 0)),
        ],
        out_specs=[
            pl.BlockSpec((r, 2), lambda i: (i, 0)),
            pl.BlockSpec((r, 2), lambda i: (i, 0)),
        ],
        out_shape=[
            jax.ShapeDtypeStruct((n, 2), jnp.int32),
            jax.ShapeDtypeStruct((n, 2), jnp.float32),
        ],
        compiler_params=pltpu.CompilerParams(
            dimension_semantics=("arbitrary",),
        ),
    )(x, W_gate, b_gate.reshape(1, e))
    return (idx, probs)


# speedup vs baseline: 1.7440x; 1.1697x over previous
import jax
import jax.numpy as jnp
from jax.experimental import pallas as pl
from jax.experimental.pallas import tpu as pltpu

_ROWS_PER_BLOCK = 1024


def _router_block(x_ref, w_ref, b_ref, idx_ref, probs_ref):
    logits = jnp.dot(x_ref[...], w_ref[...], preferred_element_type=jnp.float32)
    logits = logits + b_ref[...]
    r, e = logits.shape
    col = jax.lax.broadcasted_iota(jnp.int32, (r, e), 1)
    m1 = jnp.max(logits, axis=1, keepdims=True)
    i1 = jnp.min(jnp.where(logits == m1, col, e), axis=1, keepdims=True)
    masked = jnp.where(col == i1, -jnp.inf, logits)
    m2 = jnp.max(masked, axis=1, keepdims=True)
    i2 = jnp.min(jnp.where(masked == m2, col, e), axis=1, keepdims=True)
    idx_ref[...] = jnp.concatenate([i1, i2], axis=1)
    ex = jnp.exp(m2 - m1)
    denom = 1.0 + ex
    probs_ref[...] = jnp.concatenate([1.0 / denom, ex / denom], axis=1)


def kernel(x, W_gate, b_gate):
    n, d = x.shape
    e = W_gate.shape[1]
    r = _ROWS_PER_BLOCK
    idx, probs = pl.pallas_call(
        _router_block,
        grid=(n // r,),
        in_specs=[
            pl.BlockSpec((r, d), lambda i: (i, 0)),
            pl.BlockSpec((d, e), lambda i: (0, 0)),
            pl.BlockSpec((1, e), lambda i: (0, 0)),
        ],
        out_specs=[
            pl.BlockSpec((r, 2), lambda i: (i, 0)),
            pl.BlockSpec((r, 2), lambda i: (i, 0)),
        ],
        out_shape=[
            jax.ShapeDtypeStruct((n, 2), jnp.int32),
            jax.ShapeDtypeStruct((n, 2), jnp.float32),
        ],
        compiler_params=pltpu.CompilerParams(
            dimension_semantics=("arbitrary",),
        ),
    )(x, W_gate, b_gate.reshape(1, e))
    return (idx, probs)


# speedup vs baseline: 1.8219x; 1.0447x over previous
import jax
import jax.numpy as jnp
from jax.experimental import pallas as pl
from jax.experimental.pallas import tpu as pltpu

_ROWS_PER_BLOCK = 2048


def _router_block(x_ref, w_ref, b_ref, idx_ref, probs_ref):
    logits = jnp.dot(x_ref[...], w_ref[...], preferred_element_type=jnp.float32)
    logits = logits + b_ref[...]
    r, e = logits.shape
    col = jax.lax.broadcasted_iota(jnp.int32, (r, e), 1)
    m1 = jnp.max(logits, axis=1, keepdims=True)
    i1 = jnp.min(jnp.where(logits == m1, col, e), axis=1, keepdims=True)
    masked = jnp.where(col == i1, -jnp.inf, logits)
    m2 = jnp.max(masked, axis=1, keepdims=True)
    i2 = jnp.min(jnp.where(masked == m2, col, e), axis=1, keepdims=True)
    idx_ref[...] = jnp.concatenate([i1, i2], axis=1)
    ex = jnp.exp(m2 - m1)
    denom = 1.0 + ex
    probs_ref[...] = jnp.concatenate([1.0 / denom, ex / denom], axis=1)


def kernel(x, W_gate, b_gate):
    n, d = x.shape
    e = W_gate.shape[1]
    r = _ROWS_PER_BLOCK
    idx, probs = pl.pallas_call(
        _router_block,
        grid=(n // r,),
        in_specs=[
            pl.BlockSpec((r, d), lambda i: (i, 0)),
            pl.BlockSpec((d, e), lambda i: (0, 0)),
            pl.BlockSpec((1, e), lambda i: (0, 0)),
        ],
        out_specs=[
            pl.BlockSpec((r, 2), lambda i: (i, 0)),
            pl.BlockSpec((r, 2), lambda i: (i, 0)),
        ],
        out_shape=[
            jax.ShapeDtypeStruct((n, 2), jnp.int32),
            jax.ShapeDtypeStruct((n, 2), jnp.float32),
        ],
        compiler_params=pltpu.CompilerParams(
            dimension_semantics=("arbitrary",),
        ),
    )(x, W_gate, b_gate.reshape(1, e))
    return (idx, probs)
